# baseline (speedup 1.0000x reference)
import jax
import jax.numpy as jnp
from jax.experimental import pallas as pl

_MAXR = 32
_HEADS = 16
_S = 2048
_B = 256


def _bias_kernel(tab_ref, out_ref):
    s = pl.program_id(1)
    c = pl.program_id(2)
    diff = s - c

    @pl.when(diff >= 2)
    def _lo_const():
        out_ref[0, :, :] = jnp.full((_B, _B), tab_ref[0, 0, 2 * _MAXR - 2], jnp.float32)

    @pl.when(diff <= -2)
    def _hi_const():
        out_ref[0, :, :] = jnp.full((_B, _B), tab_ref[0, 0, 0], jnp.float32)

    @pl.when(jnp.abs(diff) <= 1)
    def _band():
        d0 = diff * _B
        ii = jax.lax.broadcasted_iota(jnp.int32, (_B, _B), 0)
        jj = jax.lax.broadcasted_iota(jnp.int32, (_B, _B), 1)
        rp = jnp.clip(d0 + ii - jj, -_MAXR + 1, _MAXR - 1) + (_MAXR - 1)
        lut = jnp.broadcast_to(tab_ref[0, 0, :], (_B, 128))
        out_ref[0, :, :] = jnp.take_along_axis(lut, rp, axis=1)


def kernel(seq_len, table):
    tab = jnp.zeros((_HEADS, 1, 128), jnp.float32)
    tab = tab.at[:, 0, : 2 * _MAXR - 1].set(table.T)
    out = pl.pallas_call(
        _bias_kernel,
        grid=(_HEADS, _S // _B, _S // _B),
        in_specs=[pl.BlockSpec((1, 1, 128), lambda h, s, c: (h, 0, 0))],
        out_specs=pl.BlockSpec((1, _B, _B), lambda h, s, c: (h, s, c)),
        out_shape=jax.ShapeDtypeStruct((_HEADS, _S, _S), jnp.float32),
    )(tab)
    return out[None]


# speedup vs baseline: 3.0835x; 3.0835x over previous
import jax
import jax.numpy as jnp
from jax.experimental import pallas as pl

_MAXR = 32
_HEADS = 16
_S = 2048
_BI = 256
_W = 128


def _bias_kernel(tab_ref, out_ref):
    s = pl.program_id(1)
    lut = tab_ref[0, 0, :]
    c_lo = jnp.full((_BI, _W), tab_ref[0, 0, 2 * _MAXR - 2], jnp.float32)
    c_hi = jnp.full((_BI, _W), tab_ref[0, 0, 0], jnp.float32)
    ij = jax.lax.broadcasted_iota(jnp.int32, (_BI, _W), 0) - jax.lax.broadcasted_iota(
        jnp.int32, (_BI, _W), 1
    )
    lut2 = jnp.broadcast_to(lut, (_BI, 128))
    s2 = s * 2
    for c in range(_S // _W):
        @pl.when(s2 - c >= 2)
        def _lo_const():
            out_ref[0, :, c * _W : (c + 1) * _W] = c_lo

        @pl.when(s2 - c <= -3)
        def _hi_const():
            out_ref[0, :, c * _W : (c + 1) * _W] = c_hi

        @pl.when(jnp.logical_and(s2 - c >= -2, s2 - c <= 1))
        def _band():
            d0 = s * _BI - c * _W
            rp = jnp.clip(ij + d0, -_MAXR + 1, _MAXR - 1) + (_MAXR - 1)
            out_ref[0, :, c * _W : (c + 1) * _W] = jnp.take_along_axis(lut2, rp, axis=1)


def kernel(seq_len, table):
    tab = jnp.zeros((_HEADS, 1, 128), jnp.float32)
    tab = tab.at[:, 0, : 2 * _MAXR - 1].set(table.T)
    out = pl.pallas_call(
        _bias_kernel,
        grid=(_HEADS, _S // _BI),
        in_specs=[pl.BlockSpec((1, 1, 128), lambda h, s: (h, 0, 0))],
        out_specs=pl.BlockSpec((1, _BI, _S), lambda h, s: (h, s, 0)),
        out_shape=jax.ShapeDtypeStruct((_HEADS, _S, _S), jnp.float32),
    )(tab)
    return out[None]


# speedup vs baseline: 3.7860x; 1.2278x over previous
import jax
import jax.numpy as jnp
from jax.experimental import pallas as pl
from jax.experimental.pallas import tpu as pltpu

_MAXR = 32
_HEADS = 16
_S = 2048
_BI = 256
_W = 128


def _bias_kernel(tab_ref, out_ref, pat_ref):
    s = pl.program_id(1)

    @pl.when(s == 0)
    def _build_pattern():
        ij = jax.lax.broadcasted_iota(jnp.int32, (_BI, _W), 0) - jax.lax.broadcasted_iota(
            jnp.int32, (_BI, _W), 1
        )
        lut2 = jnp.broadcast_to(tab_ref[0, 0, :], (_BI, 128))
        for k in range(4):
            rp = jnp.clip(ij + (128 - 128 * k), -_MAXR + 1, _MAXR - 1) + (_MAXR - 1)
            pat_ref[k, :, :] = jnp.take_along_axis(lut2, rp, axis=1)

    c_lo = jnp.full((_BI, _W), tab_ref[0, 0, 2 * _MAXR - 2], jnp.float32)
    c_hi = jnp.full((_BI, _W), tab_ref[0, 0, 0], jnp.float32)
    s2 = s * 2
    for c in range(_S // _W):
        @pl.when(s2 - c >= 2)
        def _lo_const():
            out_ref[0, :, c * _W : (c + 1) * _W] = c_lo

        @pl.when(s2 - c <= -3)
        def _hi_const():
            out_ref[0, :, c * _W : (c + 1) * _W] = c_hi

        @pl.when(jnp.logical_and(s2 - c >= -2, s2 - c <= 1))
        def _band():
            out_ref[0, :, c * _W : (c + 1) * _W] = pat_ref[c - s2 + 1, :, :]


def kernel(seq_len, table):
    tab = jnp.zeros((_HEADS, 1, 128), jnp.float32)
    tab = tab.at[:, 0, : 2 * _MAXR - 1].set(table.T)
    out = pl.pallas_call(
        _bias_kernel,
        grid=(_HEADS, _S // _BI),
        in_specs=[pl.BlockSpec((1, 1, 128), lambda h, s: (h, 0, 0))],
        out_specs=pl.BlockSpec((1, _BI, _S), lambda h, s: (h, s, 0)),
        out_shape=jax.ShapeDtypeStruct((_HEADS, _S, _S), jnp.float32),
        scratch_shapes=[pltpu.VMEM((4, _BI, _W), jnp.float32)],
    )(tab)
    return out[None]


# speedup vs baseline: 3.9292x; 1.0378x over previous
import jax
import jax.numpy as jnp
from jax.experimental import pallas as pl
from jax.experimental.pallas import tpu as pltpu

_MAXR = 32
_HEADS = 16
_S = 2048
_BI = 256
_W = 256
_NCH = 3


def _bias_kernel(tab_ref, out_ref, pat_ref):
    s = pl.program_id(1)

    @pl.when(s == 0)
    def _build_pattern():
        ij = jax.lax.broadcasted_iota(jnp.int32, (_BI, _W), 0) - jax.lax.broadcasted_iota(
            jnp.int32, (_BI, _W), 1
        )
        lut2 = jnp.broadcast_to(tab_ref[0, 0, :], (_BI, 128))
        for k in range(_NCH):
            rp = jnp.clip(ij + (_W - _W * k), -_MAXR + 1, _MAXR - 1) + (_MAXR - 1)
            pat_ref[k, :, :] = jnp.take_along_axis(lut2, rp, axis=1)

    c_lo = jnp.full((_BI, _W), tab_ref[0, 0, 2 * _MAXR - 2], jnp.float32)
    c_hi = jnp.full((_BI, _W), tab_ref[0, 0, 0], jnp.float32)
    for c in range(_S // _W):
        @pl.when(s - c >= 2)
        def _lo_const():
            out_ref[0, :, c * _W : (c + 1) * _W] = c_lo

        @pl.when(s - c <= -2)
        def _hi_const():
            out_ref[0, :, c * _W : (c + 1) * _W] = c_hi

        @pl.when(jnp.logical_and(s - c >= -1, s - c <= 1))
        def _band():
            out_ref[0, :, c * _W : (c + 1) * _W] = pat_ref[c - s + 1, :, :]


def kernel(seq_len, table):
    tab = jnp.zeros((_HEADS, 1, 128), jnp.float32)
    tab = tab.at[:, 0, : 2 * _MAXR - 1].set(table.T)
    out = pl.pallas_call(
        _bias_kernel,
        grid=(_HEADS, _S // _BI),
        in_specs=[pl.BlockSpec((1, 1, 128), lambda h, s: (h, 0, 0))],
        out_specs=pl.BlockSpec((1, _BI, _S), lambda h, s: (h, s, 0)),
        out_shape=jax.ShapeDtypeStruct((_HEADS, _S, _S), jnp.float32),
        scratch_shapes=[pltpu.VMEM((_NCH, _BI, _W), jnp.float32)],
    )(tab)
    return out[None]


# speedup vs baseline: 4.7386x; 1.2060x over previous
import jax
import jax.numpy as jnp
from jax.experimental import pallas as pl
from jax.experimental.pallas import tpu as pltpu

_MAXR = 32
_HEADS = 16
_S = 2048
_BI = 512
_W = 256
_NCH = 4


def _bias_kernel(tab_ref, out_ref, pat_ref):
    s = pl.program_id(1)

    @pl.when(s == 0)
    def _build_pattern():
        ij = jax.lax.broadcasted_iota(jnp.int32, (_BI, _W), 0) - jax.lax.broadcasted_iota(
            jnp.int32, (_BI, _W), 1
        )
        lut2 = jnp.broadcast_to(tab_ref[0, 0, :], (_BI, 128))
        for k in range(_NCH):
            rp = jnp.clip(ij + (_W - _W * k), -_MAXR + 1, _MAXR - 1) + (_MAXR - 1)
            pat_ref[k, :, :] = jnp.take_along_axis(lut2, rp, axis=1)

    c_lo = jnp.full((_BI, _W), tab_ref[0, 0, 2 * _MAXR - 2], jnp.float32)
    c_hi = jnp.full((_BI, _W), tab_ref[0, 0, 0], jnp.float32)
    s2 = s * 2
    for c in range(_S // _W):
        @pl.when(s2 - c >= 2)
        def _lo_const():
            out_ref[0, :, c * _W : (c + 1) * _W] = c_lo

        @pl.when(s2 - c <= -3)
        def _hi_const():
            out_ref[0, :, c * _W : (c + 1) * _W] = c_hi

        @pl.when(jnp.logical_and(s2 - c >= -2, s2 - c <= 1))
        def _band():
            out_ref[0, :, c * _W : (c + 1) * _W] = pat_ref[c - s2 + 1, :, :]


def kernel(seq_len, table):
    tab = jnp.zeros((_HEADS, 1, 128), jnp.float32)
    tab = tab.at[:, 0, : 2 * _MAXR - 1].set(table.T)
    out = pl.pallas_call(
        _bias_kernel,
        grid=(_HEADS, _S // _BI),
        in_specs=[pl.BlockSpec((1, 1, 128), lambda h, s: (h, 0, 0))],
        out_specs=pl.BlockSpec((1, _BI, _S), lambda h, s: (h, s, 0)),
        out_shape=jax.ShapeDtypeStruct((_HEADS, _S, _S), jnp.float32),
        scratch_shapes=[pltpu.VMEM((_NCH, _BI, _W), jnp.float32)],
    )(tab)
    return out[None]
